# baseline (speedup 1.0000x reference)
import jax
import jax.numpy as jnp
from jax.experimental import pallas as pl
from jax.experimental.pallas import tpu as pltpu

_NF = 2


def _moe_step(meta_ref, x_ref, w1_ref, b1_ref, w2_ref, b2_ref, o_ref, *, B, E):
    g = pl.program_id(0)
    f = pl.program_id(1)
    start = meta_ref[B + E + g]
    count = meta_ref[B + 2 * E + g]
    w1 = w1_ref[0]
    w2 = w2_ref[0]
    b1v = b1_ref[0, 0]
    b2v = b2_ref[0, 0]

    def body(j, carry):
        si = meta_ref[start + j]
        x = x_ref[si]
        h = jax.nn.gelu(jnp.dot(x, w1, preferred_element_type=jnp.float32) + b1v)
        part = jnp.dot(h, w2, preferred_element_type=jnp.float32)

        @pl.when(f == 0)
        def _():
            o_ref[si] = part + b2v

        @pl.when(f != 0)
        def _():
            o_ref[si] = o_ref[si] + part

        return carry

    jax.lax.fori_loop(0, count, body, 0)


def _moe_ffn(meta, hidden_states, W1, b1, W2, b2):
    B, S, D = hidden_states.shape
    E, _, F = W1.shape
    Ft = F // _NF

    import functools
    kern = functools.partial(_moe_step, B=B, E=E)

    def whole(i, f, m):
        return (0, 0, 0)

    def w1_map(g, f, m):
        return (m[B + g], 0, f)

    def b1_map(g, f, m):
        return (m[B + g], 0, f)

    def w2_map(g, f, m):
        return (m[B + g], f, 0)

    def b2_map(g, f, m):
        return (m[B + g], 0, 0)

    single = pl.Buffered(buffer_count=1)
    grid_spec = pltpu.PrefetchScalarGridSpec(
        num_scalar_prefetch=1,
        grid=(E, _NF),
        in_specs=[
            pl.BlockSpec((B, S, D), whole, pipeline_mode=single),
            pl.BlockSpec((1, D, Ft), w1_map),
            pl.BlockSpec((1, 1, Ft), b1_map),
            pl.BlockSpec((1, Ft, D), w2_map),
            pl.BlockSpec((1, 1, D), b2_map),
        ],
        out_specs=pl.BlockSpec((B, S, D), whole, pipeline_mode=single),
    )
    return pl.pallas_call(
        kern,
        grid_spec=grid_spec,
        out_shape=jax.ShapeDtypeStruct((B, S, D), jnp.float32),
    )(meta, hidden_states, W1, b1[:, None, :], W2, b2[:, None, :])


def _routing_meta(assignment, B, E):
    a = assignment.astype(jnp.int32)
    order = jnp.argsort(a).astype(jnp.int32)
    se = jnp.take(a, order)
    is_new = jnp.concatenate(
        [jnp.ones((1,), jnp.int32), (se[1:] != se[:-1]).astype(jnp.int32)])
    gid = jnp.cumsum(is_new) - 1
    n_used = gid[-1] + 1
    used = jnp.zeros((E,), jnp.int32).at[gid].set(se)
    used = jnp.where(jnp.arange(E) < n_used, used, se[-1])
    gstart = jnp.full((E,), B, jnp.int32).at[gid].min(
        jnp.arange(B, dtype=jnp.int32))
    gstart = jnp.where(jnp.arange(E) < n_used, gstart, 0)
    gcount = jnp.zeros((E,), jnp.int32).at[gid].add(1)
    gcount = jnp.where(jnp.arange(E) < n_used, gcount, 0)
    return jnp.concatenate([order, used, gstart, gcount])


def kernel(hidden_states, assignment, W1, b1, W2, b2):
    B = hidden_states.shape[0]
    E = W1.shape[0]
    meta = _routing_meta(assignment, B, E)
    return _moe_ffn(meta, hidden_states, W1, b1, W2, b2)


# speedup vs baseline: 1.0256x; 1.0256x over previous
import functools

import jax
import jax.numpy as jnp
from jax.experimental import pallas as pl
from jax.experimental.pallas import tpu as pltpu


def _moe_step(meta_ref, x_hbm, w1_ref, b1_ref, w2_ref, b2_ref, o_hbm,
              xstage, xsem, ostage, osem, *, B, E):
    g = pl.program_id(0)
    start = meta_ref[B + E + g]
    count = meta_ref[B + 2 * E + g]
    w1 = w1_ref[0]
    w2 = w2_ref[0]
    b1v = b1_ref[0, 0]
    b2v = b2_ref[0, 0]

    def issue(s):
        si = meta_ref[s]
        pltpu.make_async_copy(
            x_hbm.at[si], xstage.at[s % 2], xsem.at[s % 2]).start()

    def body(j, carry):
        s = start + j
        p = s % 2

        @pl.when(s == 0)
        def _():
            issue(s)

        pltpu.make_async_copy(x_hbm.at[0], xstage.at[p], xsem.at[p]).wait()

        @pl.when(s + 1 < B)
        def _():
            issue(s + 1)

        x = xstage[p]
        h = jax.nn.gelu(jnp.dot(x, w1, preferred_element_type=jnp.float32) + b1v)
        y = jnp.dot(h, w2, preferred_element_type=jnp.float32) + b2v

        @pl.when(s >= 2)
        def _():
            pltpu.make_async_copy(ostage.at[p], o_hbm.at[0], osem.at[p]).wait()

        ostage[p] = y
        pltpu.make_async_copy(
            ostage.at[p], o_hbm.at[meta_ref[s]], osem.at[p]).start()
        return carry

    jax.lax.fori_loop(0, count, body, 0)

    @pl.when(g == E - 1)
    def _():
        pltpu.make_async_copy(ostage.at[0], o_hbm.at[0], osem.at[0]).wait()
        pltpu.make_async_copy(ostage.at[1], o_hbm.at[0], osem.at[1]).wait()


def _moe_ffn(meta, hidden_states, W1, b1, W2, b2):
    B, S, D = hidden_states.shape
    E, _, F = W1.shape
    kern = functools.partial(_moe_step, B=B, E=E)

    def w_map(g, m):
        return (m[B + g], 0, 0)

    grid_spec = pltpu.PrefetchScalarGridSpec(
        num_scalar_prefetch=1,
        grid=(E,),
        in_specs=[
            pl.BlockSpec(memory_space=pl.MemorySpace.ANY),
            pl.BlockSpec((1, D, F), w_map),
            pl.BlockSpec((1, 1, F), w_map),
            pl.BlockSpec((1, F, D), w_map),
            pl.BlockSpec((1, 1, D), w_map),
        ],
        out_specs=pl.BlockSpec(memory_space=pl.MemorySpace.ANY),
        scratch_shapes=[
            pltpu.VMEM((2, S, D), jnp.float32),
            pltpu.SemaphoreType.DMA((2,)),
            pltpu.VMEM((2, S, D), jnp.float32),
            pltpu.SemaphoreType.DMA((2,)),
        ],
    )
    return pl.pallas_call(
        kern,
        grid_spec=grid_spec,
        out_shape=jax.ShapeDtypeStruct((B, S, D), jnp.float32),
    )(meta, hidden_states, W1, b1[:, None, :], W2, b2[:, None, :])


def _routing_meta(assignment, B, E):
    a = assignment.astype(jnp.int32)
    order = jnp.argsort(a).astype(jnp.int32)
    counts = jnp.sum((a[:, None] == jnp.arange(E, dtype=jnp.int32)[None, :])
                     .astype(jnp.int32), axis=0)
    starts = jnp.cumsum(counts) - counts
    n_used = jnp.sum((counts > 0).astype(jnp.int32))
    eids = jnp.arange(E, dtype=jnp.int32)
    used = jnp.argsort(jnp.where(counts > 0, eids, eids + E)).astype(jnp.int32)
    pad = used[n_used - 1]
    in_range = eids < n_used
    used = jnp.where(in_range, used, pad)
    gstart = jnp.where(in_range, jnp.take(starts, used), 0).astype(jnp.int32)
    gcount = jnp.where(in_range, jnp.take(counts, used), 0).astype(jnp.int32)
    return jnp.concatenate([order, used, gstart, gcount])


def kernel(hidden_states, assignment, W1, b1, W2, b2):
    B = hidden_states.shape[0]
    E = W1.shape[0]
    meta = _routing_meta(assignment, B, E)
    return _moe_ffn(meta, hidden_states, W1, b1, W2, b2)


# speedup vs baseline: 1.0421x; 1.0161x over previous
import functools

import jax
import jax.numpy as jnp
from jax.experimental import pallas as pl
from jax.experimental.pallas import tpu as pltpu


def _moe_step(meta_ref, x_hbm, w1_hbm, b1_hbm, w2_hbm, b2_hbm, o_hbm,
              xstage, w1buf, w2buf, b1buf, b2buf, ostage,
              xsem, w1sem, w2sem, b1sem, b2sem, osem, *, B, E):
    g = pl.program_id(0)
    start = meta_ref[B + E + g]
    count = meta_ref[B + 2 * E + g]
    n_used = meta_ref[B + 3 * E]

    def issue_x(s):
        pltpu.make_async_copy(
            x_hbm.at[meta_ref[s]], xstage.at[s % 2], xsem.at[s % 2]).start()

    def issue_weights(gg):
        e = meta_ref[B + gg]
        slot = gg % 2
        pltpu.make_async_copy(w1_hbm.at[e], w1buf.at[slot], w1sem.at[slot]).start()
        pltpu.make_async_copy(w2_hbm.at[e], w2buf.at[slot], w2sem.at[slot]).start()
        pltpu.make_async_copy(b1_hbm.at[e], b1buf.at[slot], b1sem.at[slot]).start()
        pltpu.make_async_copy(b2_hbm.at[e], b2buf.at[slot], b2sem.at[slot]).start()

    def wait_weights(gg):
        slot = gg % 2
        pltpu.make_async_copy(w1_hbm.at[0], w1buf.at[slot], w1sem.at[slot]).wait()
        pltpu.make_async_copy(w2_hbm.at[0], w2buf.at[slot], w2sem.at[slot]).wait()
        pltpu.make_async_copy(b1_hbm.at[0], b1buf.at[slot], b1sem.at[slot]).wait()
        pltpu.make_async_copy(b2_hbm.at[0], b2buf.at[slot], b2sem.at[slot]).wait()

    @pl.when(g == 0)
    def _():
        issue_x(0)
        issue_weights(0)

    @pl.when(g + 1 < n_used)
    def _():
        issue_weights(g + 1)

    @pl.when(g < n_used)
    def _():
        wait_weights(g)
        slot = g % 2
        w1 = w1buf[slot]
        w2 = w2buf[slot]
        b1v = b1buf[slot, 0]
        b2v = b2buf[slot, 0]

        def body(j, carry):
            s = start + j
            p = s % 2
            pltpu.make_async_copy(
                x_hbm.at[0], xstage.at[p], xsem.at[p]).wait()

            @pl.when(s + 1 < B)
            def _():
                issue_x(s + 1)

            x = xstage[p]
            h = jax.nn.gelu(
                jnp.dot(x, w1, preferred_element_type=jnp.float32) + b1v)
            y = jnp.dot(h, w2, preferred_element_type=jnp.float32) + b2v

            @pl.when(s >= 2)
            def _():
                pltpu.make_async_copy(
                    ostage.at[p], o_hbm.at[0], osem.at[p]).wait()

            ostage[p] = y
            pltpu.make_async_copy(
                ostage.at[p], o_hbm.at[meta_ref[s]], osem.at[p]).start()
            return carry

        jax.lax.fori_loop(0, count, body, 0)

    @pl.when(g == E - 1)
    def _():
        pltpu.make_async_copy(ostage.at[0], o_hbm.at[0], osem.at[0]).wait()
        pltpu.make_async_copy(ostage.at[1], o_hbm.at[0], osem.at[1]).wait()


def _moe_ffn(meta, hidden_states, W1, b1, W2, b2):
    B, S, D = hidden_states.shape
    E, _, F = W1.shape
    kern = functools.partial(_moe_step, B=B, E=E)
    anyspec = pl.BlockSpec(memory_space=pltpu.HBM)

    grid_spec = pltpu.PrefetchScalarGridSpec(
        num_scalar_prefetch=1,
        grid=(E,),
        in_specs=[anyspec] * 5,
        out_specs=anyspec,
        scratch_shapes=[
            pltpu.VMEM((2, S, D), jnp.float32),
            pltpu.VMEM((2, D, F), jnp.float32),
            pltpu.VMEM((2, F, D), jnp.float32),
            pltpu.VMEM((2, 1, F), jnp.float32),
            pltpu.VMEM((2, 1, D), jnp.float32),
            pltpu.VMEM((2, S, D), jnp.float32),
            pltpu.SemaphoreType.DMA((2,)),
            pltpu.SemaphoreType.DMA((2,)),
            pltpu.SemaphoreType.DMA((2,)),
            pltpu.SemaphoreType.DMA((2,)),
            pltpu.SemaphoreType.DMA((2,)),
            pltpu.SemaphoreType.DMA((2,)),
        ],
    )
    return pl.pallas_call(
        kern,
        grid_spec=grid_spec,
        out_shape=jax.ShapeDtypeStruct((B, S, D), jnp.float32),
    )(meta, hidden_states, W1, b1[:, None, :], W2, b2[:, None, :])


def _routing_meta(assignment, B, E):
    a = assignment.astype(jnp.int32)
    order = jnp.argsort(a).astype(jnp.int32)
    counts = jnp.sum((a[:, None] == jnp.arange(E, dtype=jnp.int32)[None, :])
                     .astype(jnp.int32), axis=0)
    starts = jnp.cumsum(counts) - counts
    n_used = jnp.sum((counts > 0).astype(jnp.int32))
    eids = jnp.arange(E, dtype=jnp.int32)
    used = jnp.argsort(jnp.where(counts > 0, eids, eids + E)).astype(jnp.int32)
    pad = used[n_used - 1]
    in_range = eids < n_used
    used = jnp.where(in_range, used, pad)
    gstart = jnp.where(in_range, jnp.take(starts, used), 0).astype(jnp.int32)
    gcount = jnp.where(in_range, jnp.take(counts, used), 0).astype(jnp.int32)
    return jnp.concatenate([order, used, gstart, gcount, n_used[None]])


def kernel(hidden_states, assignment, W1, b1, W2, b2):
    B = hidden_states.shape[0]
    E = W1.shape[0]
    meta = _routing_meta(assignment, B, E)
    return _moe_ffn(meta, hidden_states, W1, b1, W2, b2)


# speedup vs baseline: 1.3074x; 1.2546x over previous
import functools

import jax
import jax.numpy as jnp
from jax.experimental import pallas as pl
from jax.experimental.pallas import tpu as pltpu


def _moe_step(meta_ref, x_hbm, w1_hbm, b1_hbm, w2_hbm, b2_hbm, o_hbm,
              xstage, w1buf, w2buf, b1buf, b2buf, ostage,
              xsem, w1sem, w2sem, b1sem, b2sem, osem, *, B, E):
    g = pl.program_id(0)
    start = meta_ref[B + E + g]
    count = meta_ref[B + 2 * E + g]
    n_used = meta_ref[B + 3 * E]

    def issue_x(s):
        pltpu.make_async_copy(
            x_hbm.at[meta_ref[s]], xstage.at[s % 8], xsem.at[s % 8]).start()

    def issue_weights(gg):
        e = meta_ref[B + gg]
        slot = gg % 2
        pltpu.make_async_copy(w1_hbm.at[e], w1buf.at[slot], w1sem.at[slot]).start()
        pltpu.make_async_copy(w2_hbm.at[e], w2buf.at[slot], w2sem.at[slot]).start()
        pltpu.make_async_copy(b1_hbm.at[e], b1buf.at[slot], b1sem.at[slot]).start()
        pltpu.make_async_copy(b2_hbm.at[e], b2buf.at[slot], b2sem.at[slot]).start()

    def wait_weights(gg):
        slot = gg % 2
        pltpu.make_async_copy(w1_hbm.at[0], w1buf.at[slot], w1sem.at[slot]).wait()
        pltpu.make_async_copy(w2_hbm.at[0], w2buf.at[slot], w2sem.at[slot]).wait()
        pltpu.make_async_copy(b1_hbm.at[0], b1buf.at[slot], b1sem.at[slot]).wait()
        pltpu.make_async_copy(b2_hbm.at[0], b2buf.at[slot], b2sem.at[slot]).wait()

    @pl.when(g < n_used)
    def _():
        jax.lax.fori_loop(
            0, jnp.minimum(count, 8), lambda j, c: (issue_x(start + j), c)[1], 0)

    @pl.when(g == 0)
    def _():
        issue_weights(0)

    @pl.when(g + 1 < n_used)
    def _():
        issue_weights(g + 1)

    @pl.when(g < n_used)
    def _():
        wait_weights(g)
        slot = g % 2
        w1 = w1buf[slot]
        w2 = w2buf[slot]
        b1v = b1buf[slot, 0]
        b2v = b2buf[slot, 0]

        def body(j, carry):
            s = start + j
            p = s % 8
            po = s % 2
            pltpu.make_async_copy(
                x_hbm.at[0], xstage.at[p], xsem.at[p]).wait()

            @pl.when(j + 8 < count)
            def _():
                issue_x(s + 8)

            x = xstage[p]
            h = jax.nn.gelu(
                jnp.dot(x, w1, preferred_element_type=jnp.float32) + b1v)
            y = jnp.dot(h, w2, preferred_element_type=jnp.float32) + b2v

            @pl.when(s >= 2)
            def _():
                pltpu.make_async_copy(
                    ostage.at[po], o_hbm.at[0], osem.at[po]).wait()

            ostage[po] = y
            pltpu.make_async_copy(
                ostage.at[po], o_hbm.at[meta_ref[s]], osem.at[po]).start()
            return carry

        jax.lax.fori_loop(0, count, body, 0)

    @pl.when(g == E - 1)
    def _():
        pltpu.make_async_copy(ostage.at[0], o_hbm.at[0], osem.at[0]).wait()
        pltpu.make_async_copy(ostage.at[1], o_hbm.at[0], osem.at[1]).wait()


def _moe_ffn(meta, hidden_states, W1, b1, W2, b2):
    B, S, D = hidden_states.shape
    E, _, F = W1.shape
    kern = functools.partial(_moe_step, B=B, E=E)
    anyspec = pl.BlockSpec(memory_space=pltpu.HBM)

    grid_spec = pltpu.PrefetchScalarGridSpec(
        num_scalar_prefetch=1,
        grid=(E,),
        in_specs=[anyspec] * 5,
        out_specs=anyspec,
        scratch_shapes=[
            pltpu.VMEM((8, S, D), jnp.float32),
            pltpu.VMEM((2, D, F), jnp.float32),
            pltpu.VMEM((2, F, D), jnp.float32),
            pltpu.VMEM((2, 1, F), jnp.float32),
            pltpu.VMEM((2, 1, D), jnp.float32),
            pltpu.VMEM((2, S, D), jnp.float32),
            pltpu.SemaphoreType.DMA((8,)),
            pltpu.SemaphoreType.DMA((2,)),
            pltpu.SemaphoreType.DMA((2,)),
            pltpu.SemaphoreType.DMA((2,)),
            pltpu.SemaphoreType.DMA((2,)),
            pltpu.SemaphoreType.DMA((2,)),
        ],
    )
    return pl.pallas_call(
        kern,
        grid_spec=grid_spec,
        out_shape=jax.ShapeDtypeStruct((B, S, D), jnp.float32),
    )(meta, hidden_states, W1, b1[:, None, :], W2, b2[:, None, :])


def _routing_meta(assignment, B, E):
    a = assignment.astype(jnp.int32)
    order = jnp.argsort(a).astype(jnp.int32)
    counts = jnp.sum((a[:, None] == jnp.arange(E, dtype=jnp.int32)[None, :])
                     .astype(jnp.int32), axis=0)
    starts = jnp.cumsum(counts) - counts
    n_used = jnp.sum((counts > 0).astype(jnp.int32))
    eids = jnp.arange(E, dtype=jnp.int32)
    used = jnp.argsort(jnp.where(counts > 0, eids, eids + E)).astype(jnp.int32)
    pad = used[n_used - 1]
    in_range = eids < n_used
    used = jnp.where(in_range, used, pad)
    gstart = jnp.where(in_range, jnp.take(starts, used), 0).astype(jnp.int32)
    gcount = jnp.where(in_range, jnp.take(counts, used), 0).astype(jnp.int32)
    return jnp.concatenate([order, used, gstart, gcount, n_used[None]])


def kernel(hidden_states, assignment, W1, b1, W2, b2):
    B = hidden_states.shape[0]
    E = W1.shape[0]
    meta = _routing_meta(assignment, B, E)
    return _moe_ffn(meta, hidden_states, W1, b1, W2, b2)
